# baseline (speedup 1.0000x reference)
import functools

import jax
import jax.numpy as jnp
from jax import lax
from jax.experimental import pallas as pl
from jax.experimental.pallas import tpu as pltpu
from jax.experimental.pallas import tpu_sc as plsc

_B, _T, _S = 16, 2048, 512
_NC, _NS, _L = 2, 16, 16
_NW = _NC * _NS
_B1 = 8
_NB_SC = _B - _B1
_RPW = _NB_SC * _T // _NW
_WPB = _T // _RPW
_CHUNK = 32
_NCHUNK = _RPW // _CHUNK
_TB = 256


def _sc_body(src_hbm, shifts_hbm, out_hbm, shifts_v,
             inb0, inb1, outb0, outb1, si0, si1, so0, so1):
    wid = lax.axis_index("s") * _NC + lax.axis_index("c")
    b = _B1 + wid // _WPB
    t0 = (wid % _WPB) * _RPW
    pltpu.sync_copy(shifts_hbm.at[b, pl.ds(t0, _RPW)], shifts_v)

    iota = lax.iota(jnp.int32, _L)
    zero16 = iota * 0

    def in_chunk(g):
        return src_hbm.at[b, pl.ds(t0 + g * _CHUNK, _CHUNK), :]

    def out_chunk(g):
        return out_hbm.at[b - _B1, pl.ds(t0 + g * _CHUNK, _CHUNK), :]

    def start_in(g, ib, si):
        gc = jnp.minimum(g, _NCHUNK - 1)
        pltpu.async_copy(in_chunk(gc), ib, si)

    def start_out(g, ob, so):
        pltpu.async_copy(ob, out_chunk(g), so)

    def compute(g, ib, ob):
        @plsc.parallel_loop(0, _CHUNK, step=1, unroll=4)
        def row_body(r):
            ridx = g * _CHUNK + r
            shift_vec = plsc.load_gather(shifts_v, [zero16 + ridx])
            idx0 = (iota - shift_vec) & (_S - 1)
            rvec = zero16 + r
            for j in range(_S // _L):
                elem = (idx0 + (_L * j)) & (_S - 1)
                vec = plsc.load_gather(ib, [rvec, elem])
                ob[r, pl.ds(_L * j, _L)] = vec

    start_in(0, inb0, si0)
    start_in(1, inb1, si1)

    def pair_body(k, carry):
        for g_off, (ib, ob, si, so) in enumerate(
            ((inb0, outb0, si0, so0), (inb1, outb1, si1, so1))):
            g = 2 * k + g_off
            pltpu.make_async_copy(in_chunk(0), ib, si).wait()

            @pl.when(k > 0)
            def _():
                pltpu.make_async_copy(ob, out_chunk(0), so).wait()

            compute(g, ib, ob)
            start_out(g, ob, so)
            start_in(g + 2, ib, si)
        return carry

    lax.fori_loop(0, _NCHUNK // 2, pair_body, 0)

    pltpu.make_async_copy(in_chunk(0), inb0, si0).wait()
    pltpu.make_async_copy(in_chunk(0), inb1, si1).wait()
    pltpu.make_async_copy(outb0, out_chunk(0), so0).wait()
    pltpu.make_async_copy(outb1, out_chunk(0), so1).wait()


def _sc_roll(src, shifts_i32):
    mesh = plsc.VectorSubcoreMesh(core_axis_name="c", subcore_axis_name="s")
    return pl.kernel(
        _sc_body,
        out_type=jax.ShapeDtypeStruct((_NB_SC, _T, _S), jnp.float32),
        mesh=mesh,
        compiler_params=pltpu.CompilerParams(needs_layout_passes=False),
        scratch_types=(
            [pltpu.VMEM((_RPW,), jnp.int32)]
            + [pltpu.VMEM((_CHUNK, _S), jnp.float32) for _ in range(4)]
            + [pltpu.SemaphoreType.DMA for _ in range(4)]
        ),
    )(src, shifts_i32)


def _tc_body(src_ref, shift_ref, out_ref):
    x = src_ref[0]
    bit_src = shift_ref[0, 0][:, None]
    for k in range(9):
        bit = ((bit_src >> k) & 1) == 1
        rolled = pltpu.roll(x, 1 << k, 1)
        x = jnp.where(bit, rolled, x)
    out_ref[0] = x


def _tc_roll(src, shifts_i32):
    sh3 = shifts_i32.reshape(_B * _T // _TB, 1, _TB)
    nj = _T // _TB
    return pl.pallas_call(
        _tc_body,
        grid=(_B1, nj),
        in_specs=[pl.BlockSpec((1, _TB, _S), lambda i, j: (i, j, 0)),
                  pl.BlockSpec((1, 1, _TB), lambda i, j: (i * nj + j, 0, 0))],
        out_specs=pl.BlockSpec((1, _TB, _S), lambda i, j: (i, j, 0)),
        out_shape=jax.ShapeDtypeStruct((_B1, _T, _S), jnp.float32),
    )(src, sh3)


@jax.jit
def kernel(src, shifts):
    shifts_i32 = shifts.astype(jnp.int32)
    out_sc = _sc_roll(src, shifts_i32)
    out_tc = _tc_roll(src, shifts_i32)
    return jnp.concatenate([out_tc, out_sc], axis=0)


# speedup vs baseline: 1.8460x; 1.8460x over previous
import functools

import jax
import jax.numpy as jnp
from jax import lax
from jax.experimental import pallas as pl
from jax.experimental.pallas import tpu as pltpu
from jax.experimental.pallas import tpu_sc as plsc

_B, _T, _S = 16, 2048, 512
_NROWS = _B * _T
_NC, _NS, _L = 2, 16, 16
_NW = _NC * _NS
_ROWS_PER_W = _NROWS // _NW
_CHUNK = 32
_NCHUNK = _ROWS_PER_W // _CHUNK
_NIN = 2
_NOUT = 4


def _roll_body(src_hbm, shifts_hbm, out_hbm, shifts_v, *bufs_and_sems):
    inbs = bufs_and_sems[0:_NIN]
    outbs = bufs_and_sems[_NIN:_NIN + _NOUT]
    sis = bufs_and_sems[_NIN + _NOUT:2 * _NIN + _NOUT]
    sos = bufs_and_sems[2 * _NIN + _NOUT:]

    wid = lax.axis_index("s") * _NC + lax.axis_index("c")
    b = wid // 2
    t0 = (wid % 2) * _ROWS_PER_W
    pltpu.sync_copy(shifts_hbm.at[b, pl.ds(t0, _ROWS_PER_W)], shifts_v)

    iota = lax.iota(jnp.int32, _L)
    zero16 = iota * 0

    def hbm_chunk(ref, g):
        return ref.at[b, pl.ds(t0 + g * _CHUNK, _CHUNK), :]

    def start_in(g, ib, si):
        gc = jnp.minimum(g, _NCHUNK - 1)
        pltpu.async_copy(hbm_chunk(src_hbm, gc), ib, si)

    def start_out(g, ob, so):
        pltpu.async_copy(ob, hbm_chunk(out_hbm, g), so)

    def compute(g, ib, ob):
        @plsc.parallel_loop(0, _CHUNK, step=1, unroll=4)
        def row_body(r):
            ridx = g * _CHUNK + r
            shift_vec = plsc.load_gather(shifts_v, [zero16 + ridx])
            idx0 = (iota - shift_vec) & (_S - 1)
            rvec = zero16 + r
            for j in range(_S // _L):
                elem = (idx0 + (_L * j)) & (_S - 1)
                vec = plsc.load_gather(ib, [rvec, elem])
                ob[r, pl.ds(_L * j, _L)] = vec

    for i in range(_NIN):
        start_in(i, inbs[i], sis[i])

    def ring_body(k, carry):
        for off in range(_NOUT):
            g = _NOUT * k + off
            ib, si = inbs[off % _NIN], sis[off % _NIN]
            ob, so = outbs[off], sos[off]
            pltpu.make_async_copy(hbm_chunk(src_hbm, 0), ib, si).wait()

            @pl.when(k > 0)
            def _():
                pltpu.make_async_copy(ob, hbm_chunk(out_hbm, 0), so).wait()

            compute(g, ib, ob)
            start_out(g, ob, so)
            start_in(g + _NIN, ib, si)
        return carry

    lax.fori_loop(0, _NCHUNK // _NOUT, ring_body, 0)

    for i in range(_NIN):
        pltpu.make_async_copy(hbm_chunk(src_hbm, 0), inbs[i], sis[i]).wait()
    for i in range(_NOUT):
        pltpu.make_async_copy(outbs[i], hbm_chunk(out_hbm, 0), sos[i]).wait()


@jax.jit
def kernel(src, shifts):
    shifts_i32 = shifts.astype(jnp.int32)
    mesh = plsc.VectorSubcoreMesh(core_axis_name="c", subcore_axis_name="s")
    return pl.kernel(
        _roll_body,
        out_type=jax.ShapeDtypeStruct((_B, _T, _S), jnp.float32),
        mesh=mesh,
        compiler_params=pltpu.CompilerParams(needs_layout_passes=False),
        scratch_types=(
            [pltpu.VMEM((_ROWS_PER_W,), jnp.int32)]
            + [pltpu.VMEM((_CHUNK, _S), jnp.float32) for _ in range(_NIN + _NOUT)]
            + [pltpu.SemaphoreType.DMA for _ in range(_NIN + _NOUT)]
        ),
    )(src, shifts_i32)


# speedup vs baseline: 1.8478x; 1.0010x over previous
import functools

import jax
import jax.numpy as jnp
from jax import lax
from jax.experimental import pallas as pl
from jax.experimental.pallas import tpu as pltpu
from jax.experimental.pallas import tpu_sc as plsc

_B, _T, _S = 16, 2048, 512
_NROWS = _B * _T
_NC, _NS, _L = 2, 16, 16
_NW = _NC * _NS
_ROWS_PER_W = _NROWS // _NW
_CHUNK = 32
_NCHUNK = _ROWS_PER_W // _CHUNK
_NIN = 2
_NOUT = 4


def _roll_body(src_hbm, shifts_hbm, out_hbm, shifts_v, *bufs_and_sems):
    inbs = bufs_and_sems[0:_NIN]
    outbs = bufs_and_sems[_NIN:_NIN + _NOUT]
    sis = bufs_and_sems[_NIN + _NOUT:2 * _NIN + _NOUT]
    sos = bufs_and_sems[2 * _NIN + _NOUT:]

    wid = lax.axis_index("s") * _NC + lax.axis_index("c")
    b = wid // 2
    t0 = (wid % 2) * _ROWS_PER_W
    pltpu.sync_copy(shifts_hbm.at[b, pl.ds(t0, _ROWS_PER_W)], shifts_v)

    iota = lax.iota(jnp.int32, _L)
    zero16 = iota * 0

    def hbm_chunk(ref, g):
        return ref.at[b, pl.ds(t0 + g * _CHUNK, _CHUNK), :]

    def start_in(g, ib, si):
        gc = jnp.minimum(g, _NCHUNK - 1)
        pltpu.async_copy(hbm_chunk(src_hbm, gc), ib, si)

    def start_out(g, ob, so):
        pltpu.async_copy(ob, hbm_chunk(out_hbm, g), so)

    def compute(g, ib, ob):
        @plsc.parallel_loop(0, _CHUNK, step=1, unroll=4)
        def row_body(r):
            ridx = g * _CHUNK + r
            shift_vec = plsc.load_gather(shifts_v, [zero16 + ridx])
            idx0 = (iota - shift_vec) & (_S - 1)
            rvec = zero16 + r
            for j in range(_S // _L):
                col = (idx0 + (_L * j)) & (_S - 1)
                vec = plsc.load_gather(ib, [rvec, col])
                ob[r, pl.ds(_L * j, _L)] = vec

    for i in range(_NIN):
        start_in(i, inbs[i], sis[i])

    def ring_body(k, carry):
        for off in range(_NOUT):
            g = _NOUT * k + off
            ib, si = inbs[off % _NIN], sis[off % _NIN]
            ob, so = outbs[off], sos[off]
            pltpu.make_async_copy(hbm_chunk(src_hbm, 0), ib, si).wait()

            @pl.when(k > 0)
            def _():
                pltpu.make_async_copy(ob, hbm_chunk(out_hbm, 0), so).wait()

            compute(g, ib, ob)
            start_out(g, ob, so)
            start_in(g + _NIN, ib, si)
        return carry

    lax.fori_loop(0, _NCHUNK // _NOUT, ring_body, 0)

    for i in range(_NIN):
        pltpu.make_async_copy(hbm_chunk(src_hbm, 0), inbs[i], sis[i]).wait()
    for i in range(_NOUT):
        pltpu.make_async_copy(outbs[i], hbm_chunk(out_hbm, 0), sos[i]).wait()


@jax.jit
def kernel(src, shifts):
    shifts_i32 = shifts.astype(jnp.int32)
    mesh = plsc.VectorSubcoreMesh(core_axis_name="c", subcore_axis_name="s")
    return pl.kernel(
        _roll_body,
        out_type=jax.ShapeDtypeStruct((_B, _T, _S), jnp.float32),
        mesh=mesh,
        compiler_params=pltpu.CompilerParams(needs_layout_passes=False),
        scratch_types=(
            [pltpu.VMEM((_ROWS_PER_W,), jnp.int32)]
            + [pltpu.VMEM((_CHUNK, _S), jnp.float32) for _ in range(_NIN + _NOUT)]
            + [pltpu.SemaphoreType.DMA for _ in range(_NIN + _NOUT)]
        ),
    )(src, shifts_i32)
